# baseline (speedup 1.0000x reference)
import functools

import jax
import jax.numpy as jnp
from jax import lax
from jax.experimental import pallas as pl
from jax.experimental.pallas import tpu as pltpu
from jax.experimental.pallas import tpu_sc as plsc

_D = 768
_NP = 576
_NT = _NP + 1
_B = 64
_NC, _NS = 2, 16
_L = 16
_C = 16
_K = _NP // _C
_CPR = _D // _L


@functools.partial(
    pl.kernel,
    out_type=jax.ShapeDtypeStruct((_B, _NT, _D), jnp.float32),
    mesh=plsc.VectorSubcoreMesh(core_axis_name="c", subcore_axis_name="s"),
    scratch_types=[
        pltpu.VMEM_SHARED((_NT, _D), jnp.float32),
        pltpu.VMEM((_C, _D), jnp.float32),
        pltpu.VMEM((_C, _D), jnp.float32),
        pltpu.VMEM((_C, _D), jnp.float32),
        pltpu.VMEM((_C, _D), jnp.float32),
        pltpu.VMEM((_C, _D), jnp.float32),
        pltpu.VMEM((_C, _D), jnp.float32),
        pltpu.VMEM((1, _D), jnp.float32),
        pltpu.VMEM((1, _D), jnp.float32),
        pltpu.VMEM((8, _D), jnp.float32),
        pltpu.VMEM((8, _D), jnp.float32),
        pltpu.SemaphoreType.DMA,
        pltpu.SemaphoreType.DMA,
        pltpu.SemaphoreType.DMA,
        pltpu.SemaphoreType.DMA,
        pltpu.SemaphoreType.DMA,
        pltpu.SemaphoreType.DMA,
    ],
)
def _sc_kernel(in_hbm, cls_hbm, pos_hbm, out_hbm,
               spos, in0_v, in1_v, po0_v, po1_v, out0_v, out1_v,
               cls_v, carry_v, ptail_v, otail_v,
               isem0, isem1, psem0, psem1, osem0, osem1):
    wid = lax.axis_index("s") * _NC + lax.axis_index("c")
    in_bufs = (in0_v, in1_v)
    po_bufs = (po0_v, po1_v)
    out_bufs = (out0_v, out1_v)
    in_sems = (isem0, isem1)
    po_sems = (psem0, psem1)
    out_sems = (osem0, osem1)

    @pl.when(lax.axis_index("s") == 0)
    def _():
        pltpu.sync_copy(pos_hbm, spos)

    plsc.subcore_barrier()

    pltpu.sync_copy(cls_hbm, cls_v)
    t0 = pl.multiple_of(_NP + (wid - wid), 8)
    pltpu.sync_copy(spos.at[pl.ds(t0, 8)], ptail_v)

    def in_copy(s, b, k):
        r0 = pl.multiple_of(_C * k, 8)
        return pltpu.make_async_copy(
            in_hbm.at[b, pl.ds(r0, _C), :], in_bufs[s], in_sems[s])

    def pos_copy(s, k):
        r0 = pl.multiple_of(_C * k, 8)
        return pltpu.make_async_copy(pos_hbm.at[pl.ds(r0, _C)], po_bufs[s], po_sems[s])

    def out_copy(s, b, k):
        r0 = pl.multiple_of(_C * k, 8)
        return pltpu.make_async_copy(
            out_bufs[s], out_hbm.at[b, pl.ds(r0, _C), :], out_sems[s])

    for o in (0, 1):
        b = wid * 2 + o

        in_copy(0, b, 0).start()
        pos_copy(0, 0).start()
        in_copy(1, b, 1).start()
        pos_copy(1, 1).start()

        def pair_body(i, carry, b=b):
            for s in (0, 1):
                k = 2 * i + s
                in_copy(s, b, k).wait()
                pos_copy(s, k).wait()

                @pl.when(i >= 1)
                def _():
                    out_copy(s, b, k - 2).wait()

                if s == 0:
                    @pl.when(i == 0)
                    def _():
                        for c in range(_CPR):
                            d = pl.ds(c * _L, _L)
                            out_bufs[0][0, d] = cls_v[0, d] + po_bufs[0][0, d]

                    @pl.when(i > 0)
                    def _():
                        for c in range(_CPR):
                            d = pl.ds(c * _L, _L)
                            out_bufs[0][0, d] = carry_v[0, d] + po_bufs[0][0, d]
                else:
                    for c in range(_CPR):
                        d = pl.ds(c * _L, _L)
                        out_bufs[1][0, d] = carry_v[0, d] + po_bufs[1][0, d]

                def row_body(r, rc):
                    for c in range(_CPR):
                        d = pl.ds(c * _L, _L)
                        out_bufs[s][r, d] = in_bufs[s][r - 1, d] + po_bufs[s][r, d]
                    return rc

                lax.fori_loop(1, _C, row_body, 0)

                for c in range(_CPR):
                    d = pl.ds(c * _L, _L)
                    carry_v[0, d] = in_bufs[s][_C - 1, d]

                out_copy(s, b, k).start()

                @pl.when(i < _K // 2 - 1)
                def _():
                    in_copy(s, b, k + 2).start()
                    pos_copy(s, k + 2).start()
            return carry

        lax.fori_loop(0, _K // 2, pair_body, 0)

        for c in range(_CPR):
            d = pl.ds(c * _L, _L)
            otail_v[0, d] = carry_v[0, d] + ptail_v[0, d]
        pltpu.sync_copy(otail_v, out_hbm.at[b, pl.ds(t0, 8), :])

        out_copy(0, b, _K - 2).wait()
        out_copy(1, b, _K - 1).wait()


def kernel(inputs, class_embed, pos_table):
    return _sc_kernel(inputs, class_embed.reshape(1, _D), pos_table)


# speedup vs baseline: 1.0246x; 1.0246x over previous
import functools

import jax
import jax.numpy as jnp
from jax import lax
from jax.experimental import pallas as pl
from jax.experimental.pallas import tpu as pltpu
from jax.experimental.pallas import tpu_sc as plsc

_D = 768
_NP = 576
_NT = _NP + 1
_B = 64
_NC, _NS = 2, 16
_L = 16
_C = 24
_K = _NP // _C
_CPR = _D // _L


@functools.partial(
    pl.kernel,
    out_type=jax.ShapeDtypeStruct((_B, _NT, _D), jnp.float32),
    mesh=plsc.VectorSubcoreMesh(core_axis_name="c", subcore_axis_name="s"),
    scratch_types=[
        pltpu.VMEM((_C, _D), jnp.float32),
        pltpu.VMEM((_C, _D), jnp.float32),
        pltpu.VMEM((_C, _D), jnp.float32),
        pltpu.VMEM((_C, _D), jnp.float32),
        pltpu.VMEM((_C, _D), jnp.float32),
        pltpu.VMEM((_C, _D), jnp.float32),
        pltpu.VMEM((1, _D), jnp.float32),
        pltpu.VMEM((1, _D), jnp.float32),
        pltpu.VMEM((8, _D), jnp.float32),
        pltpu.VMEM((8, _D), jnp.float32),
        pltpu.SemaphoreType.DMA,
        pltpu.SemaphoreType.DMA,
        pltpu.SemaphoreType.DMA,
        pltpu.SemaphoreType.DMA,
        pltpu.SemaphoreType.DMA,
        pltpu.SemaphoreType.DMA,
    ],
)
def _sc_kernel(in_hbm, cls_hbm, pos_hbm, out_hbm,
               in0_v, in1_v, po0_v, po1_v, out0_v, out1_v,
               cls_v, carry_v, ptail_v, otail_v,
               isem0, isem1, psem0, psem1, osem0, osem1):
    wid = lax.axis_index("s") * _NC + lax.axis_index("c")
    in_bufs = (in0_v, in1_v)
    po_bufs = (po0_v, po1_v)
    out_bufs = (out0_v, out1_v)
    in_sems = (isem0, isem1)
    po_sems = (psem0, psem1)
    out_sems = (osem0, osem1)

    pltpu.sync_copy(cls_hbm, cls_v)
    t0 = pl.multiple_of(_NP + (wid - wid), 8)
    pltpu.sync_copy(pos_hbm.at[pl.ds(t0, 8)], ptail_v)

    def in_copy(s, b, k):
        r0 = pl.multiple_of(_C * k, 8)
        return pltpu.make_async_copy(
            in_hbm.at[b, pl.ds(r0, _C), :], in_bufs[s], in_sems[s])

    def pos_copy(s, k):
        r0 = pl.multiple_of(_C * k, 8)
        return pltpu.make_async_copy(pos_hbm.at[pl.ds(r0, _C)], po_bufs[s], po_sems[s])

    def out_copy(s, b, k):
        r0 = pl.multiple_of(_C * k, 8)
        return pltpu.make_async_copy(
            out_bufs[s], out_hbm.at[b, pl.ds(r0, _C), :], out_sems[s])

    for o in (0, 1):
        b = wid * 2 + o

        in_copy(0, b, 0).start()
        pos_copy(0, 0).start()
        in_copy(1, b, 1).start()
        pos_copy(1, 1).start()

        def pair_body(i, carry, b=b):
            for s in (0, 1):
                k = 2 * i + s
                in_copy(s, b, k).wait()
                pos_copy(s, k).wait()

                @pl.when(i >= 1)
                def _():
                    out_copy(s, b, k - 2).wait()

                if s == 0:
                    @pl.when(i == 0)
                    def _():
                        for c in range(_CPR):
                            d = pl.ds(c * _L, _L)
                            out_bufs[0][0, d] = cls_v[0, d] + po_bufs[0][0, d]

                    @pl.when(i > 0)
                    def _():
                        for c in range(_CPR):
                            d = pl.ds(c * _L, _L)
                            out_bufs[0][0, d] = carry_v[0, d] + po_bufs[0][0, d]
                else:
                    for c in range(_CPR):
                        d = pl.ds(c * _L, _L)
                        out_bufs[1][0, d] = carry_v[0, d] + po_bufs[1][0, d]

                def row_body(r, rc):
                    for c in range(_CPR):
                        d = pl.ds(c * _L, _L)
                        out_bufs[s][r, d] = in_bufs[s][r - 1, d] + po_bufs[s][r, d]
                    return rc

                lax.fori_loop(1, _C, row_body, 0)

                for c in range(_CPR):
                    d = pl.ds(c * _L, _L)
                    carry_v[0, d] = in_bufs[s][_C - 1, d]

                out_copy(s, b, k).start()

                @pl.when(i < _K // 2 - 1)
                def _():
                    in_copy(s, b, k + 2).start()
                    pos_copy(s, k + 2).start()
            return carry

        lax.fori_loop(0, _K // 2, pair_body, 0)

        for c in range(_CPR):
            d = pl.ds(c * _L, _L)
            otail_v[0, d] = carry_v[0, d] + ptail_v[0, d]
        pltpu.sync_copy(otail_v, out_hbm.at[b, pl.ds(t0, 8), :])

        out_copy(0, b, _K - 2).wait()
        out_copy(1, b, _K - 1).wait()


def kernel(inputs, class_embed, pos_table):
    return _sc_kernel(inputs, class_embed.reshape(1, _D), pos_table)
